# baseline (speedup 1.0000x reference)
import functools

import jax
import jax.numpy as jnp
from jax import lax
from jax.experimental import pallas as pl
from jax.experimental.pallas import tpu as pltpu
from jax.experimental.pallas import tpu_sc as plsc

B = 128
N = 32768
NC = 2
NS = 16
L = 16
NW = NC * NS
B_SC = 48
B_TC = B - B_SC
C = N // 2
HC = 2 * B_SC // NW
U = 8
RB = 16

_mesh = plsc.VectorSubcoreMesh(core_axis_name="c", subcore_axis_name="s")


@functools.partial(
    pl.kernel,
    mesh=_mesh,
    out_type=jax.ShapeDtypeStruct((B_SC, 2 * L), jnp.float32),
    scratch_types=[
        pltpu.VMEM((C,), jnp.float32),
        pltpu.VMEM((C,), jnp.float32),
        pltpu.VMEM((C,), jnp.float32),
        pltpu.VMEM((C,), jnp.float32),
        pltpu.VMEM((HC, L), jnp.float32),
        pltpu.SemaphoreType.DMA,
        pltpu.SemaphoreType.DMA,
        pltpu.SemaphoreType.DMA,
        pltpu.SemaphoreType.DMA,
    ],
)
def _hinge_sc(x_hbm, y_hbm, out_hbm, xb0, xb1, yb0, yb1, ob,
              sx0, sx1, sy0, sy1):
    cid = lax.axis_index("c")
    sid = lax.axis_index("s")
    wid = sid * NC + cid
    j0 = HC * wid

    def start(j, xb, yb, sx, sy):
        r = j // 2
        off = (j % 2) * C
        pltpu.async_copy(x_hbm.at[r, pl.ds(off, C)], xb, sx)
        pltpu.async_copy(y_hbm.at[r, pl.ds(off, C)], yb, sy)

    def wait(xb, yb, sx, sy):
        pltpu.make_async_copy(x_hbm.at[0, pl.ds(0, C)], xb, sx).wait()
        pltpu.make_async_copy(y_hbm.at[0, pl.ds(0, C)], yb, sy).wait()

    def make_chunk(xb, yb):
        def body(i, ms):
            out = []
            for u in range(U):
                xv = xb[pl.ds(i + u * L, L)]
                yv = yb[pl.ds(i + u * L, L)]
                out.append(jnp.maximum(ms[u], jnp.where(yv >= 0.5, xv, 0.0)))
            return tuple(out)
        return body

    def tree_max(ms):
        m = ms[0]
        for u in range(1, U):
            m = jnp.maximum(m, ms[u])
        return m

    init = tuple(jnp.full((L,), -jnp.inf, jnp.float32) for _ in range(U))
    xbufs, ybufs = (xb0, xb1), (yb0, yb1)
    sxs, sys_ = (sx0, sx1), (sy0, sy1)

    start(j0, xb0, yb0, sx0, sy0)
    start(j0 + 1, xb1, yb1, sx1, sy1)

    for jj in range(HC):
        s = jj % 2
        wait(xbufs[s], ybufs[s], sxs[s], sys_[s])
        ms = plsc.parallel_loop(0, C, L * U, unroll=4, carry=init)(
            make_chunk(xbufs[s], ybufs[s]))
        if jj + 2 < HC:
            start(j0 + jj + 2, xbufs[s], ybufs[s], sxs[s], sys_[s])
        ob[jj, :] = tree_max(ms)

    for jj in range(HC):
        j = j0 + jj
        pltpu.sync_copy(ob.at[jj],
                        out_hbm.at[j // 2, pl.ds((j % 2) * L, L)])


def _rowmax_tc(x_ref, y_ref, o_ref):
    s = jnp.where(y_ref[...] >= 0.5, x_ref[...], 0.0)
    o_ref[...] = jnp.max(s, axis=1, keepdims=True)


_tcmax = pl.pallas_call(
    _rowmax_tc,
    grid=(B_TC // RB,),
    in_specs=[
        pl.BlockSpec((RB, N), lambda i: (i + B_SC // RB, 0)),
        pl.BlockSpec((RB, N), lambda i: (i + B_SC // RB, 0)),
    ],
    out_specs=pl.BlockSpec((RB, 1), lambda i: (i, 0)),
    out_shape=jax.ShapeDtypeStruct((B_TC, 1), jnp.float32),
)


def _combine_tc(sc_ref, tc_ref, o_ref):
    sc_sum = jnp.sum(jnp.max(sc_ref[...], axis=1))
    tc_sum = jnp.sum(tc_ref[...])
    o_ref[0, 0] = (sc_sum + tc_sum) * (1.0 / B)


_combine = pl.pallas_call(
    _combine_tc,
    out_shape=jax.ShapeDtypeStruct((1, 1), jnp.float32),
    out_specs=pl.BlockSpec(memory_space=pltpu.SMEM),
)


@jax.jit
def kernel(x, y):
    sc_partials = _hinge_sc(x, y)
    tc_max = _tcmax(x, y)
    return _combine(sc_partials, tc_max)[0, 0]


# speedup vs baseline: 1.0133x; 1.0133x over previous
import functools

import jax
import jax.numpy as jnp
from jax import lax
from jax.experimental import pallas as pl
from jax.experimental.pallas import tpu as pltpu
from jax.experimental.pallas import tpu_sc as plsc

B = 128
N = 32768
NC = 2
NS = 16
L = 16
NW = NC * NS
B_SC = 48
B_TC = B - B_SC
C = N // 4
PR = N // C
HC = PR * B_SC // NW
U = 8
RB = 8

_mesh = plsc.VectorSubcoreMesh(core_axis_name="c", subcore_axis_name="s")


@functools.partial(
    pl.kernel,
    mesh=_mesh,
    out_type=jax.ShapeDtypeStruct((B_SC, PR * L), jnp.float32),
    scratch_types=[
        pltpu.VMEM((C,), jnp.float32),
        pltpu.VMEM((C,), jnp.float32),
        pltpu.VMEM((C,), jnp.float32),
        pltpu.VMEM((C,), jnp.float32),
        pltpu.VMEM((HC, L), jnp.float32),
        pltpu.SemaphoreType.DMA,
        pltpu.SemaphoreType.DMA,
        pltpu.SemaphoreType.DMA,
        pltpu.SemaphoreType.DMA,
    ],
)
def _hinge_sc(x_hbm, y_hbm, out_hbm, xb0, xb1, yb0, yb1, ob,
              sx0, sx1, sy0, sy1):
    cid = lax.axis_index("c")
    sid = lax.axis_index("s")
    wid = sid * NC + cid
    j0 = HC * wid

    def start(j, xb, yb, sx, sy):
        r = j // PR
        off = (j % PR) * C
        pltpu.async_copy(x_hbm.at[r, pl.ds(off, C)], xb, sx)
        pltpu.async_copy(y_hbm.at[r, pl.ds(off, C)], yb, sy)

    def wait(xb, yb, sx, sy):
        pltpu.make_async_copy(x_hbm.at[0, pl.ds(0, C)], xb, sx).wait()
        pltpu.make_async_copy(y_hbm.at[0, pl.ds(0, C)], yb, sy).wait()

    def make_chunk(xb, yb):
        def body(i, ms):
            out = []
            for u in range(U):
                xv = xb[pl.ds(i + u * L, L)]
                yv = yb[pl.ds(i + u * L, L)]
                out.append(jnp.maximum(ms[u], jnp.where(yv >= 0.5, xv, 0.0)))
            return tuple(out)
        return body

    def tree_max(ms):
        m = ms[0]
        for u in range(1, U):
            m = jnp.maximum(m, ms[u])
        return m

    init = tuple(jnp.full((L,), -jnp.inf, jnp.float32) for _ in range(U))
    xbufs, ybufs = (xb0, xb1), (yb0, yb1)
    sxs, sys_ = (sx0, sx1), (sy0, sy1)

    start(j0, xb0, yb0, sx0, sy0)
    start(j0 + 1, xb1, yb1, sx1, sy1)

    for jj in range(HC):
        s = jj % 2
        wait(xbufs[s], ybufs[s], sxs[s], sys_[s])
        ms = plsc.parallel_loop(0, C, L * U, unroll=4, carry=init)(
            make_chunk(xbufs[s], ybufs[s]))
        if jj + 2 < HC:
            start(j0 + jj + 2, xbufs[s], ybufs[s], sxs[s], sys_[s])
        ob[jj, :] = tree_max(ms)

    for jj in range(HC):
        j = j0 + jj
        pltpu.sync_copy(ob.at[jj],
                        out_hbm.at[j // PR, pl.ds((j % PR) * L, L)])


def _rowmax_tc(x_ref, y_ref, o_ref):
    s = jnp.where(y_ref[...] >= 0.5, x_ref[...], 0.0)
    o_ref[...] = jnp.max(s, axis=1, keepdims=True)


_tcmax = pl.pallas_call(
    _rowmax_tc,
    grid=(B_TC // RB,),
    in_specs=[
        pl.BlockSpec((RB, N), lambda i: (i + B_SC // RB, 0)),
        pl.BlockSpec((RB, N), lambda i: (i + B_SC // RB, 0)),
    ],
    out_specs=pl.BlockSpec((RB, 1), lambda i: (i, 0)),
    out_shape=jax.ShapeDtypeStruct((B_TC, 1), jnp.float32),
)


def _combine_tc(sc_ref, tc_ref, o_ref):
    sc_sum = jnp.sum(jnp.max(sc_ref[...], axis=1))
    tc_sum = jnp.sum(tc_ref[...])
    o_ref[0, 0] = (sc_sum + tc_sum) * (1.0 / B)


_combine = pl.pallas_call(
    _combine_tc,
    out_shape=jax.ShapeDtypeStruct((1, 1), jnp.float32),
    out_specs=pl.BlockSpec(memory_space=pltpu.SMEM),
)


@jax.jit
def kernel(x, y):
    sc_partials = _hinge_sc(x, y)
    tc_max = _tcmax(x, y)
    return _combine(sc_partials, tc_max)[0, 0]
